# baseline (speedup 1.0000x reference)
import jax
import jax.numpy as jnp
from jax.experimental import pallas as pl
from jax.experimental.pallas import tpu as pltpu

_CH = 4096


def _body(m_ref, f_ref, o_ref, acc_ref, cnt_ref):
    j = pl.program_id(1)
    nj = pl.num_programs(1)

    @pl.when(j == 0)
    def _init():
        acc_ref[...] = jnp.zeros_like(acc_ref)
        cnt_ref[0] = 0.0

    m = m_ref[0, 0]
    f = f_ref[0]
    acc_ref[...] += jax.lax.dot_general(
        m, f, (((1,), (0,)), ((), ())),
        preferred_element_type=jnp.float32)
    cnt_ref[0] += jnp.sum(m)

    @pl.when(j == nj - 1)
    def _final():
        o_ref[...] = acc_ref[...][None] / jnp.maximum(cnt_ref[0], 1.0)


def kernel(features, mask):
    B, S, D = features.shape
    nch = S // _CH
    maskf = mask.astype(jnp.float32).reshape(B, nch, 1, _CH)
    out = pl.pallas_call(
        _body,
        grid=(B, nch),
        in_specs=[
            pl.BlockSpec((1, 1, 1, _CH), lambda i, j: (i, j, 0, 0)),
            pl.BlockSpec((1, _CH, D), lambda i, j: (i, j, 0)),
        ],
        out_specs=pl.BlockSpec((1, 1, D), lambda i, j: (i, 0, 0)),
        out_shape=jax.ShapeDtypeStruct((B, 1, D), jnp.float32),
        scratch_shapes=[
            pltpu.VMEM((1, D), jnp.float32),
            pltpu.SMEM((1,), jnp.float32),
        ],
        compiler_params=pltpu.CompilerParams(
            dimension_semantics=("parallel", "arbitrary"),
        ),
    )(maskf, features)
    return out.reshape(B, D)


# speedup vs baseline: 1.0637x; 1.0637x over previous
import jax
import jax.numpy as jnp
from jax.experimental import pallas as pl
from jax.experimental.pallas import tpu as pltpu

_CH = 2048


def _body(m_ref, f_ref, o_ref, acc_ref, cnt_ref):
    j = pl.program_id(1)
    nj = pl.num_programs(1)

    @pl.when(j == 0)
    def _init():
        acc_ref[...] = jnp.zeros_like(acc_ref)
        cnt_ref[0] = 0.0

    m = m_ref[pl.program_id(0), j]
    f = f_ref[0]
    acc_ref[...] += jax.lax.dot_general(
        m, f, (((1,), (0,)), ((), ())),
        preferred_element_type=jnp.float32)
    cnt_ref[0] += jnp.sum(m)

    @pl.when(j == nj - 1)
    def _final():
        o_ref[...] = acc_ref[...][None] / jnp.maximum(cnt_ref[0], 1.0)


def kernel(features, mask):
    B, S, D = features.shape
    nch = S // _CH
    maskf = mask.astype(jnp.float32).reshape(B, nch, 1, _CH)
    out = pl.pallas_call(
        _body,
        grid=(B, nch),
        in_specs=[
            pl.BlockSpec((B, nch, 1, _CH), lambda i, j: (0, 0, 0, 0)),
            pl.BlockSpec((1, _CH, D), lambda i, j: (i, j, 0)),
        ],
        out_specs=pl.BlockSpec((1, 1, D), lambda i, j: (i, 0, 0)),
        out_shape=jax.ShapeDtypeStruct((B, 1, D), jnp.float32),
        scratch_shapes=[
            pltpu.VMEM((1, D), jnp.float32),
            pltpu.SMEM((1,), jnp.float32),
        ],
        compiler_params=pltpu.CompilerParams(
            dimension_semantics=("parallel", "arbitrary"),
        ),
    )(maskf, features)
    return out.reshape(B, D)
